# baseline (speedup 1.0000x reference)
import functools

import jax
import jax.numpy as jnp
from jax import lax
from jax.experimental import pallas as pl
from jax.experimental.pallas import tpu as pltpu
from jax.experimental.pallas import tpu_sc as plsc

BATCH = 1024
CONTEXT = 20
EMBED = 64
IDX_CHUNK = 80


def _sc_gather_mean(idx2d, embeddings):
    info = plsc.get_sparse_core_info()
    nc, ns = info.num_cores, info.num_subcores
    nw = nc * ns
    b_per_w = BATCH // nw
    rows_per_w = b_per_w * CONTEXT
    chunks_per_w = rows_per_w // IDX_CHUNK

    mesh = plsc.VectorSubcoreMesh(core_axis_name="c", subcore_axis_name="s")

    @functools.partial(
        pl.kernel,
        mesh=mesh,
        out_type=jax.ShapeDtypeStruct((BATCH, EMBED), jnp.float32),
        scratch_types=[
            pltpu.VMEM((chunks_per_w, IDX_CHUNK), jnp.int32),
            pltpu.VMEM((rows_per_w, EMBED), jnp.float32),
            pltpu.VMEM((b_per_w, EMBED), jnp.float32),
            pltpu.SemaphoreType.DMA,
        ],
        compiler_params=pltpu.CompilerParams(use_tc_tiling_on_sc=False),
    )
    def sc_kernel(idx_hbm, emb_hbm, out_hbm, idx_v, rows_v, acc_v, sem):
        wid = lax.axis_index("s") * nc + lax.axis_index("c")
        pltpu.sync_copy(idx_hbm.at[pl.ds(wid * chunks_per_w, chunks_per_w)],
                        idx_v)
        copies = []
        for j in range(chunks_per_w):
            copies.append(pltpu.async_copy(
                emb_hbm.at[idx_v.at[j]],
                rows_v.at[pl.ds(j * IDX_CHUNK, IDX_CHUNK)],
                sem))
        for c in copies:
            c.wait()

        inv = jnp.float32(1.0 / CONTEXT)

        def body(b, carry):
            row0 = b * CONTEXT
            for d in range(EMBED // 16):
                sl = pl.ds(d * 16, 16)
                acc = rows_v[row0, sl]
                for t in range(1, CONTEXT):
                    acc = acc + rows_v[row0 + t, sl]
                acc_v[b, sl] = acc * inv
            return carry

        lax.fori_loop(0, b_per_w, body, 0)
        pltpu.sync_copy(acc_v, out_hbm.at[pl.ds(wid * b_per_w, b_per_w)])

    return sc_kernel(idx2d, embeddings)


def _tc_matmul(ctx, linear_w, linear_b, vb=2048):
    v = linear_w.shape[0]
    grid = pl.cdiv(v, vb)

    def mm_body(x_ref, w_ref, b_ref, o_ref):
        o_ref[...] = lax.dot_general(
            x_ref[...], w_ref[...],
            (((1,), (1,)), ((), ())),
            preferred_element_type=jnp.float32,
        ) + b_ref[...][None, :]

    return pl.pallas_call(
        mm_body,
        grid=(grid,),
        in_specs=[
            pl.BlockSpec((BATCH, EMBED), lambda j: (0, 0)),
            pl.BlockSpec((vb, EMBED), lambda j: (j, 0)),
            pl.BlockSpec((vb,), lambda j: (j,)),
        ],
        out_specs=pl.BlockSpec((BATCH, vb), lambda j: (0, j)),
        out_shape=jax.ShapeDtypeStruct((BATCH, v), jnp.float32),
    )(ctx, linear_w, linear_b)


def kernel(context_words, embeddings, linear_w, linear_b):
    idx2d = context_words.astype(jnp.int32).reshape(-1, IDX_CHUNK)
    ctx = _sc_gather_mean(idx2d, embeddings)
    return _tc_matmul(ctx, linear_w, linear_b)


# speedup vs baseline: 2.7497x; 2.7497x over previous
import functools

import jax
import jax.numpy as jnp
from jax import lax
from jax.experimental import pallas as pl
from jax.experimental.pallas import tpu as pltpu
from jax.experimental.pallas import tpu_sc as plsc

BATCH = 1024
CONTEXT = 20
EMBED = 64
IDX_CHUNK = 80


def _sc_gather_mean(idx2d, embeddings):
    info = plsc.get_sparse_core_info()
    nc, ns = info.num_cores, info.num_subcores
    nw = nc * ns
    b_per_w = BATCH // nw
    rows_per_w = b_per_w * CONTEXT
    chunks_per_w = rows_per_w // IDX_CHUNK

    mesh = plsc.VectorSubcoreMesh(core_axis_name="c", subcore_axis_name="s")

    @functools.partial(
        pl.kernel,
        mesh=mesh,
        out_type=jax.ShapeDtypeStruct((BATCH, EMBED), jnp.float32),
        scratch_types=[
            pltpu.VMEM((chunks_per_w, IDX_CHUNK), jnp.int32),
            pltpu.VMEM((rows_per_w, EMBED), jnp.float32),
            pltpu.VMEM((b_per_w, EMBED), jnp.float32),
            pltpu.SemaphoreType.DMA,
        ],
        compiler_params=pltpu.CompilerParams(use_tc_tiling_on_sc=False),
    )
    def sc_kernel(idx_hbm, emb_hbm, out_hbm, idx_v, rows_v, acc_v, sem):
        wid = lax.axis_index("s") * nc + lax.axis_index("c")
        pltpu.sync_copy(idx_hbm.at[pl.ds(wid * chunks_per_w, chunks_per_w)],
                        idx_v)
        copies = []
        for j in range(chunks_per_w):
            copies.append(pltpu.async_copy(
                emb_hbm.at[idx_v.at[j]],
                rows_v.at[pl.ds(j * IDX_CHUNK, IDX_CHUNK)],
                sem))
        for c in copies:
            c.wait()

        inv = jnp.float32(1.0 / CONTEXT)

        def body(b, carry):
            row0 = b * CONTEXT
            for d in range(EMBED // 16):
                sl = pl.ds(d * 16, 16)
                acc = rows_v[row0, sl]
                for t in range(1, CONTEXT):
                    acc = acc + rows_v[row0 + t, sl]
                acc_v[b, sl] = acc * inv
            return carry

        lax.fori_loop(0, b_per_w, body, 0)
        pltpu.sync_copy(acc_v, out_hbm.at[pl.ds(wid * b_per_w, b_per_w)])

    return sc_kernel(idx2d, embeddings)


def _tc_matmul_t(ctx, w_t, linear_b, vb=2048):
    v = w_t.shape[1]
    grid = pl.cdiv(v, vb)

    def mm_body(w_ref, x_ref, b_ref, o_ref):
        o_ref[...] = lax.dot_general(
            w_ref[...], x_ref[...],
            (((0,), (1,)), ((), ())),
            preferred_element_type=jnp.float32,
        ) + b_ref[...][:, None]

    return pl.pallas_call(
        mm_body,
        grid=(grid,),
        in_specs=[
            pl.BlockSpec((EMBED, vb), lambda j: (0, j)),
            pl.BlockSpec((BATCH, EMBED), lambda j: (0, 0)),
            pl.BlockSpec((vb,), lambda j: (j,)),
        ],
        out_specs=pl.BlockSpec((vb, BATCH), lambda j: (j, 0)),
        out_shape=jax.ShapeDtypeStruct((v, BATCH), jnp.float32),
    )(w_t, ctx, linear_b)


def kernel(context_words, embeddings, linear_w, linear_b):
    idx2d = context_words.astype(jnp.int32).reshape(-1, IDX_CHUNK)
    ctx = _sc_gather_mean(idx2d, embeddings)
    out_t = _tc_matmul_t(ctx, linear_w.T, linear_b)
    return out_t.T


# speedup vs baseline: 2.7621x; 1.0045x over previous
import functools

import jax
import jax.numpy as jnp
from jax import lax
from jax.experimental import pallas as pl
from jax.experimental.pallas import tpu as pltpu
from jax.experimental.pallas import tpu_sc as plsc

BATCH = 1024
CONTEXT = 20
EMBED = 64
IDX_CHUNK = 80


def _sc_gather_mean(idx2d, embeddings):
    info = plsc.get_sparse_core_info()
    nc, ns = info.num_cores, info.num_subcores
    nw = nc * ns
    b_per_w = BATCH // nw
    rows_per_w = b_per_w * CONTEXT
    chunks_per_w = rows_per_w // IDX_CHUNK

    mesh = plsc.VectorSubcoreMesh(core_axis_name="c", subcore_axis_name="s")

    @functools.partial(
        pl.kernel,
        mesh=mesh,
        out_type=jax.ShapeDtypeStruct((BATCH, EMBED), jnp.float32),
        scratch_types=[
            pltpu.VMEM((chunks_per_w, IDX_CHUNK), jnp.int32),
            pltpu.VMEM((rows_per_w, EMBED), jnp.float32),
            pltpu.VMEM((b_per_w, EMBED), jnp.float32),
            pltpu.SemaphoreType.DMA,
        ],
        compiler_params=pltpu.CompilerParams(use_tc_tiling_on_sc=False),
    )
    def sc_kernel(idx_hbm, emb_hbm, out_hbm, idx_v, rows_v, acc_v, sem):
        wid = lax.axis_index("s") * nc + lax.axis_index("c")
        pltpu.sync_copy(idx_hbm.at[pl.ds(wid * chunks_per_w, chunks_per_w)],
                        idx_v)
        copies = []
        for j in range(chunks_per_w):
            copies.append(pltpu.async_copy(
                emb_hbm.at[idx_v.at[j]],
                rows_v.at[pl.ds(j * IDX_CHUNK, IDX_CHUNK)],
                sem))
        for c in copies:
            c.wait()

        inv = jnp.float32(1.0 / CONTEXT)

        def body(b, carry):
            row0 = b * CONTEXT
            for d in range(EMBED // 16):
                sl = pl.ds(d * 16, 16)
                acc = rows_v[row0, sl]
                for t in range(1, CONTEXT):
                    acc = acc + rows_v[row0 + t, sl]
                acc_v[b, sl] = acc * inv
            return carry

        lax.fori_loop(0, b_per_w, body, 0)
        pltpu.sync_copy(acc_v, out_hbm.at[pl.ds(wid * b_per_w, b_per_w)])

    return sc_kernel(idx2d, embeddings)


def _tc_matmul_t(ctx, w_t, linear_b, vb=2048):
    v = w_t.shape[1]
    grid = pl.cdiv(v, vb)

    def mm_body(w_ref, x_ref, b_ref, o_ref):
        o_ref[...] = lax.dot_general(
            w_ref[...], x_ref[...],
            (((0,), (1,)), ((), ())),
            preferred_element_type=jnp.float32,
        ) + b_ref[...][:, None]

    return pl.pallas_call(
        mm_body,
        grid=(grid,),
        in_specs=[
            pl.BlockSpec((EMBED, vb), lambda j: (0, j)),
            pl.BlockSpec((BATCH, EMBED), lambda j: (0, 0)),
            pl.BlockSpec((vb,), lambda j: (j,)),
        ],
        out_specs=pl.BlockSpec((vb, BATCH), lambda j: (j, 0)),
        out_shape=jax.ShapeDtypeStruct((v, BATCH), jnp.float32),
    )(w_t, ctx, linear_b)


def kernel(context_words, embeddings, linear_w, linear_b):
    idx2d = context_words.astype(jnp.int32).reshape(-1, IDX_CHUNK)
    ctx = _sc_gather_mean(idx2d, embeddings)
    out_t = _tc_matmul_t(ctx, linear_w.T, linear_b, vb=4096)
    return out_t.T
